# baseline (speedup 1.0000x reference)
import jax
import jax.numpy as jnp
from jax.experimental import pallas as pl
from jax.experimental.pallas import tpu as pltpu

TOP_K = 2
TS = 128


def _fold_kernel(idx_ref, scores_ref, p0_ref, p1_ref, reexp_ref, out_ref):
    b = pl.program_id(0)
    i0 = idx_ref[b, 0]
    i1 = idx_ref[b, 1]
    s0 = scores_ref[b, i0]
    s1 = scores_ref[b, i1]
    m = jnp.maximum(s0, s1)
    e0 = jnp.exp(s0 - m)
    e1 = jnp.exp(s1 - m)
    denom = e0 + e1
    w0 = e0 / denom
    w1 = e1 / denom
    merged = p0_ref[0, 0] * w0 + p1_ref[0, 0] * w1
    for p in range(out_ref.shape[1]):
        out_ref[0, p] = merged + reexp_ref[p]


def kernel(probes, scores, re_expand):
    B, P, S, D = probes.shape
    _, top_idx = jax.lax.top_k(scores, TOP_K)
    top_idx = top_idx.astype(jnp.int32)

    grid = (B, S // TS)

    def probe_spec(k):
        def imap(b, s, idx_ref, scr_ref):
            return (b, idx_ref[b, k], s, 0)
        return pl.BlockSpec((1, 1, TS, D), imap)

    out_spec = pl.BlockSpec((1, P, TS, D), lambda b, s, idx_ref, scr_ref: (b, 0, s, 0))
    reexp_spec = pl.BlockSpec((P, D), lambda b, s, idx_ref, scr_ref: (0, 0))

    grid_spec = pltpu.PrefetchScalarGridSpec(
        num_scalar_prefetch=2,
        grid=grid,
        in_specs=[probe_spec(0), probe_spec(1), reexp_spec],
        out_specs=out_spec,
    )

    return pl.pallas_call(
        _fold_kernel,
        grid_spec=grid_spec,
        out_shape=jax.ShapeDtypeStruct((B, P, S, D), probes.dtype),
        compiler_params=pltpu.CompilerParams(
            dimension_semantics=("parallel", "arbitrary"),
        ),
    )(top_idx, scores, probes, probes, re_expand)


# speedup vs baseline: 1.0932x; 1.0932x over previous
import jax
import jax.numpy as jnp
from jax.experimental import pallas as pl
from jax.experimental.pallas import tpu as pltpu

TOP_K = 2
TS = 512


def _fold_kernel(idx_ref, scores_ref, p0_ref, p1_ref, reexp_ref, out_ref):
    b = pl.program_id(0)
    i0 = idx_ref[b, 0]
    i1 = idx_ref[b, 1]
    s0 = scores_ref[b, i0]
    s1 = scores_ref[b, i1]
    m = jnp.maximum(s0, s1)
    e0 = jnp.exp(s0 - m)
    e1 = jnp.exp(s1 - m)
    denom = e0 + e1
    w0 = e0 / denom
    w1 = e1 / denom
    merged = p0_ref[0, 0] * w0 + p1_ref[0, 0] * w1
    for p in range(out_ref.shape[1]):
        out_ref[0, p] = merged + reexp_ref[p]


def kernel(probes, scores, re_expand):
    B, P, S, D = probes.shape
    _, top_idx = jax.lax.top_k(scores, TOP_K)
    top_idx = top_idx.astype(jnp.int32)

    grid = (B, S // TS)

    def probe_spec(k):
        def imap(b, s, idx_ref, scr_ref):
            return (b, idx_ref[b, k], s, 0)
        return pl.BlockSpec((1, 1, TS, D), imap)

    out_spec = pl.BlockSpec((1, P, TS, D), lambda b, s, idx_ref, scr_ref: (b, 0, s, 0))
    reexp_spec = pl.BlockSpec((P, D), lambda b, s, idx_ref, scr_ref: (0, 0))

    grid_spec = pltpu.PrefetchScalarGridSpec(
        num_scalar_prefetch=2,
        grid=grid,
        in_specs=[probe_spec(0), probe_spec(1), reexp_spec],
        out_specs=out_spec,
    )

    return pl.pallas_call(
        _fold_kernel,
        grid_spec=grid_spec,
        out_shape=jax.ShapeDtypeStruct((B, P, S, D), probes.dtype),
        compiler_params=pltpu.CompilerParams(
            dimension_semantics=("parallel", "arbitrary"),
        ),
    )(top_idx, scores, probes, probes, re_expand)
